# Initial kernel scaffold; baseline (speedup 1.0000x reference)
import jax
import jax.numpy as jnp
from jax.experimental import pallas as pl


def kernel(x, edge_index, W1, b1, W2, b2):
    raise NotImplementedError("write your pallas kernel here")


# speedup vs baseline: 7.4503x; 7.4503x over previous
import functools

import jax
import jax.numpy as jnp
from jax import lax
from jax.experimental import pallas as pl
from jax.experimental.pallas import tpu as pltpu
from jax.experimental.pallas import tpu_sc as plsc

N = 10000
IN_DIM = 256
HID_DIM = 512
OUT_DIM = 256
E = 160000
HALF = 128

NTILES = 16
NPAD = 10240
ROWS_PER_TILE = NPAD // NTILES
EPAD = 163840
CHUNK = 128
NCHUNKS = EPAD // CHUNK
AGG_CHUNKS_PER_TILE = NCHUNKS // NTILES
DEG_CHUNKS_PER_TILE = NCHUNKS // (2 * NTILES)

ROW_BLOCK = 1000
GRID = N // ROW_BLOCK

_mesh = plsc.VectorSubcoreMesh(core_axis_name="c", subcore_axis_name="s")


@functools.partial(
    pl.kernel,
    out_type=jax.ShapeDtypeStruct((2 * NPAD, HALF), jnp.float32),
    mesh=_mesh,
    scratch_types=[
        pltpu.VMEM_SHARED((NPAD, HALF), jnp.float32),
        pltpu.VMEM((1, CHUNK), jnp.int32),
        pltpu.VMEM((CHUNK, HALF), jnp.float32),
    ],
)
def _deg_kernel(dst_hbm, zeros_hbm, ones_hbm, out_hbm, acc, idx_v, ones_v):
    cid = lax.axis_index("c")
    sid = lax.axis_index("s")
    row0 = sid * ROWS_PER_TILE
    pltpu.sync_copy(zeros_hbm, acc.at[pl.ds(row0, ROWS_PER_TILE)])
    pltpu.sync_copy(ones_hbm, ones_v)
    plsc.subcore_barrier()
    base = (cid * NTILES + sid) * DEG_CHUNKS_PER_TILE

    def body(g, carry):
        pltpu.sync_copy(dst_hbm.at[pl.ds(base + g, 1)], idx_v)
        pltpu.sync_copy(ones_v, acc.at[idx_v.at[0]], add=True)
        return carry

    lax.fori_loop(0, DEG_CHUNKS_PER_TILE, body, 0)
    plsc.subcore_barrier()
    pltpu.sync_copy(acc.at[pl.ds(row0, ROWS_PER_TILE)],
                    out_hbm.at[pl.ds(cid * NPAD + row0, ROWS_PER_TILE)])


@functools.partial(
    pl.kernel,
    out_type=jax.ShapeDtypeStruct((2 * NPAD, HALF), jnp.float32),
    mesh=_mesh,
    scratch_types=[
        pltpu.VMEM_SHARED((NPAD, HALF), jnp.float32),
        pltpu.VMEM((1, CHUNK), jnp.int32),
        pltpu.VMEM((1, CHUNK), jnp.int32),
        pltpu.VMEM((CHUNK, HALF), jnp.float32),
        pltpu.SemaphoreType.DMA,
    ],
)
def _agg_kernel(y_hbm, srcoff_hbm, dst_hbm, zeros_hbm, z_hbm,
                acc, idx_s, idx_d, rows, sem):
    cid = lax.axis_index("c")
    sid = lax.axis_index("s")
    row0 = sid * ROWS_PER_TILE
    pltpu.sync_copy(zeros_hbm, acc.at[pl.ds(row0, ROWS_PER_TILE)])
    plsc.subcore_barrier()
    base = cid * NCHUNKS + sid * AGG_CHUNKS_PER_TILE

    def body(g, carry):
        pltpu.sync_copy(srcoff_hbm.at[pl.ds(base + g, 1)], idx_s)
        pltpu.sync_copy(dst_hbm.at[pl.ds(sid * AGG_CHUNKS_PER_TILE + g, 1)],
                        idx_d)
        pltpu.async_copy(y_hbm.at[idx_s.at[0]], rows, sem).wait()
        pltpu.sync_copy(rows, acc.at[idx_d.at[0]], add=True)
        return carry

    lax.fori_loop(0, AGG_CHUNKS_PER_TILE, body, 0)
    plsc.subcore_barrier()
    pltpu.sync_copy(acc.at[pl.ds(row0, ROWS_PER_TILE)],
                    z_hbm.at[pl.ds(cid * NPAD + row0, ROWS_PER_TILE)])


def _rsqrt_deg(deg_ref):
    return lax.rsqrt(deg_ref[0, :, :1] + deg_ref[1, :, :1] + 1.0)


def _prep_body(deg_ref, x_ref, y_ref):
    d = _rsqrt_deg(deg_ref)
    y = x_ref[...] * d
    y_ref[0] = y[:, :HALF]
    y_ref[1] = y[:, HALF:]


def _dense_body(z_ref, y_ref, deg_ref, w1_ref, b1_ref, w2_ref, q_ref):
    d = _rsqrt_deg(deg_ref)
    a = jnp.concatenate(
        [z_ref[0] + y_ref[0], z_ref[1] + y_ref[1]], axis=1) * d
    h = jnp.maximum(
        jnp.dot(a, w1_ref[...], preferred_element_type=jnp.float32)
        + b1_ref[...], 0.0)
    p = jnp.dot(h, w2_ref[...], preferred_element_type=jnp.float32)
    q = p * d
    q_ref[0] = q[:, :HALF]
    q_ref[1] = q[:, HALF:]


def _final_body(s_ref, q_ref, deg_ref, b2_ref, out_ref):
    d = _rsqrt_deg(deg_ref)
    o = jnp.concatenate(
        [s_ref[0] + q_ref[0], s_ref[1] + q_ref[1]], axis=1) * d
    o = o + b2_ref[...]
    m = jnp.max(o, axis=1, keepdims=True)
    e = jnp.exp(o - m)
    out_ref[...] = (o - m) - jnp.log(jnp.sum(e, axis=1, keepdims=True))


def _pair_spec(width):
    return pl.BlockSpec((2, ROW_BLOCK, width), lambda i: (0, i, 0))


def _row_spec(width):
    return pl.BlockSpec((ROW_BLOCK, width), lambda i: (i, 0))


def _full_spec(shape):
    return pl.BlockSpec(shape, lambda i: (0,) * len(shape))


_PAIR128 = jax.ShapeDtypeStruct((2, N, HALF), jnp.float32)

_prep = pl.pallas_call(
    _prep_body,
    grid=(GRID,),
    in_specs=[_pair_spec(HALF), _row_spec(IN_DIM)],
    out_specs=_pair_spec(HALF),
    out_shape=_PAIR128,
)

_dense = pl.pallas_call(
    _dense_body,
    grid=(GRID,),
    in_specs=[
        _pair_spec(HALF),
        _pair_spec(HALF),
        _pair_spec(HALF),
        _full_spec((IN_DIM, HID_DIM)),
        _full_spec((1, HID_DIM)),
        _full_spec((HID_DIM, OUT_DIM)),
    ],
    out_specs=_pair_spec(HALF),
    out_shape=_PAIR128,
)

_final = pl.pallas_call(
    _final_body,
    grid=(GRID,),
    in_specs=[
        _pair_spec(HALF),
        _pair_spec(HALF),
        _pair_spec(HALF),
        _full_spec((1, OUT_DIM)),
    ],
    out_specs=_row_spec(OUT_DIM),
    out_shape=jax.ShapeDtypeStruct((N, OUT_DIM), jnp.float32),
)


def kernel(x, edge_index, W1, b1, W2, b2):
    src = edge_index[0]
    dst = edge_index[1]
    pad = EPAD - E
    src2d = jnp.concatenate(
        [src, jnp.zeros((pad,), jnp.int32)]).reshape(NCHUNKS, CHUNK)
    srcoff = jnp.concatenate([src2d, src2d + N], axis=0)
    dst2d = jnp.concatenate(
        [dst, jnp.full((pad,), N, jnp.int32)]).reshape(NCHUNKS, CHUNK)

    zeros128 = jnp.zeros((ROWS_PER_TILE, HALF), jnp.float32)
    ones128 = jnp.ones((CHUNK, HALF), jnp.float32)

    deg = _deg_kernel(dst2d, zeros128, ones128).reshape(2, NPAD, HALF)
    y = _prep(deg, x)
    y_flat = y.reshape(2 * N, HALF)
    z = _agg_kernel(y_flat, srcoff, dst2d, zeros128).reshape(2, NPAD, HALF)
    q = _dense(z, y, deg, W1, b1.reshape(1, HID_DIM), W2)
    q_flat = q.reshape(2 * N, HALF)
    s = _agg_kernel(q_flat, srcoff, dst2d, zeros128).reshape(2, NPAD, HALF)
    return _final(s, q, deg, b2.reshape(1, OUT_DIM))


# speedup vs baseline: 9.6624x; 1.2969x over previous
import functools

import jax
import jax.numpy as jnp
from jax import lax
from jax.experimental import pallas as pl
from jax.experimental.pallas import tpu as pltpu
from jax.experimental.pallas import tpu_sc as plsc

N = 10000
IN_DIM = 256
HID_DIM = 512
OUT_DIM = 256
E = 160000
HALF = 128

NTILES = 16
NPAD = 10240
ROWS_PER_TILE = NPAD // NTILES
EPAD = 163840
CHUNK = 128
NCHUNKS = EPAD // CHUNK
AGG_CHUNKS_PER_TILE = NCHUNKS // NTILES
DEG_CHUNKS_PER_TILE = NCHUNKS // (2 * NTILES)

ROW_BLOCK = 1000
GRID = N // ROW_BLOCK

_mesh = plsc.VectorSubcoreMesh(core_axis_name="c", subcore_axis_name="s")


@functools.partial(
    pl.kernel,
    out_type=jax.ShapeDtypeStruct((2 * NPAD, HALF), jnp.float32),
    mesh=_mesh,
    scratch_types=[
        pltpu.VMEM_SHARED((NPAD, HALF), jnp.float32),
        pltpu.VMEM((DEG_CHUNKS_PER_TILE, CHUNK), jnp.int32),
        pltpu.VMEM((CHUNK, HALF), jnp.float32),
    ],
)
def _deg_kernel(dst_hbm, zeros_hbm, ones_hbm, out_hbm, acc, idx_v, ones_v):
    cid = lax.axis_index("c")
    sid = lax.axis_index("s")
    row0 = sid * ROWS_PER_TILE
    pltpu.sync_copy(zeros_hbm, acc.at[pl.ds(row0, ROWS_PER_TILE)])
    base = (cid * NTILES + sid) * DEG_CHUNKS_PER_TILE
    pltpu.sync_copy(dst_hbm.at[pl.ds(base, DEG_CHUNKS_PER_TILE)], idx_v)
    pltpu.sync_copy(ones_hbm, ones_v)
    plsc.subcore_barrier()

    def body(g, carry):
        pltpu.sync_copy(ones_v, acc.at[idx_v.at[g]], add=True)
        return carry

    lax.fori_loop(0, DEG_CHUNKS_PER_TILE, body, 0)
    plsc.subcore_barrier()
    pltpu.sync_copy(acc.at[pl.ds(row0, ROWS_PER_TILE)],
                    out_hbm.at[pl.ds(cid * NPAD + row0, ROWS_PER_TILE)])


@functools.partial(
    pl.kernel,
    out_type=jax.ShapeDtypeStruct((2 * NPAD, HALF), jnp.float32),
    mesh=_mesh,
    scratch_types=[
        pltpu.VMEM_SHARED((NPAD, HALF), jnp.float32),
        pltpu.VMEM((AGG_CHUNKS_PER_TILE // 2, CHUNK), jnp.int32),
        pltpu.VMEM((AGG_CHUNKS_PER_TILE // 2, CHUNK), jnp.int32),
        pltpu.VMEM((CHUNK, HALF), jnp.float32),
        pltpu.VMEM((CHUNK, HALF), jnp.float32),
        pltpu.SemaphoreType.DMA,
        pltpu.SemaphoreType.DMA,
        pltpu.SemaphoreType.DMA,
        pltpu.SemaphoreType.DMA,
    ],
)
def _agg_kernel(y_hbm, srcoff_hbm, dst_hbm, zeros_hbm, z_hbm,
                acc, idx_s, idx_d, rows0, rows1, gs0, gs1, ss0, ss1):
    cid = lax.axis_index("c")
    sid = lax.axis_index("s")
    row0 = sid * ROWS_PER_TILE
    phase_len = AGG_CHUNKS_PER_TILE // 2
    pltpu.sync_copy(zeros_hbm, acc.at[pl.ds(row0, ROWS_PER_TILE)])
    plsc.subcore_barrier()

    def gather(g, rows, sem):
        pltpu.async_copy(y_hbm.at[idx_s.at[g]], rows, sem)

    def scatter(g, rows, sem):
        pltpu.async_copy(rows, acc.at[idx_d.at[g]], sem, add=True)

    def gwait(g, rows, sem):
        pltpu.make_async_copy(y_hbm.at[idx_s.at[g]], rows, sem).wait()

    def swait(g, rows, sem):
        pltpu.make_async_copy(rows, acc.at[idx_d.at[g]], sem).wait()

    for phase in range(2):
        pltpu.sync_copy(
            srcoff_hbm.at[pl.ds(cid * NCHUNKS + sid * AGG_CHUNKS_PER_TILE
                                + phase * phase_len, phase_len)], idx_s)
        pltpu.sync_copy(
            dst_hbm.at[pl.ds(sid * AGG_CHUNKS_PER_TILE + phase * phase_len,
                             phase_len)], idx_d)
        gather(0, rows0, gs0)
        gather(1, rows1, gs1)

        def body(i, carry):
            g = 2 * i
            gwait(g, rows0, gs0)
            scatter(g, rows0, ss0)
            gwait(g + 1, rows1, gs1)
            scatter(g + 1, rows1, ss1)

            @pl.when(i < phase_len // 2 - 1)
            def _prefetch():
                swait(g, rows0, ss0)
                gather(g + 2, rows0, gs0)
                swait(g + 1, rows1, ss1)
                gather(g + 3, rows1, gs1)

            return carry

        lax.fori_loop(0, phase_len // 2, body, 0)
        swait(phase_len - 2, rows0, ss0)
        swait(phase_len - 1, rows1, ss1)
    plsc.subcore_barrier()
    pltpu.sync_copy(acc.at[pl.ds(row0, ROWS_PER_TILE)],
                    z_hbm.at[pl.ds(cid * NPAD + row0, ROWS_PER_TILE)])


def _rsqrt_deg(deg_ref):
    return lax.rsqrt(deg_ref[0, :, :1] + deg_ref[1, :, :1] + 1.0)


def _prep_body(deg_ref, x_ref, y_ref):
    d = _rsqrt_deg(deg_ref)
    y = x_ref[...] * d
    y_ref[0] = y[:, :HALF]
    y_ref[1] = y[:, HALF:]


def _dense_body(z_ref, y_ref, deg_ref, w1_ref, b1_ref, w2_ref, q_ref):
    d = _rsqrt_deg(deg_ref)
    a = jnp.concatenate(
        [z_ref[0] + y_ref[0], z_ref[1] + y_ref[1]], axis=1) * d
    h = jnp.maximum(
        jnp.dot(a, w1_ref[...], preferred_element_type=jnp.float32)
        + b1_ref[...], 0.0)
    p = jnp.dot(h, w2_ref[...], preferred_element_type=jnp.float32)
    q = p * d
    q_ref[0] = q[:, :HALF]
    q_ref[1] = q[:, HALF:]


def _final_body(s_ref, q_ref, deg_ref, b2_ref, out_ref):
    d = _rsqrt_deg(deg_ref)
    o = jnp.concatenate(
        [s_ref[0] + q_ref[0], s_ref[1] + q_ref[1]], axis=1) * d
    o = o + b2_ref[...]
    m = jnp.max(o, axis=1, keepdims=True)
    e = jnp.exp(o - m)
    out_ref[...] = (o - m) - jnp.log(jnp.sum(e, axis=1, keepdims=True))


def _pair_spec(width):
    return pl.BlockSpec((2, ROW_BLOCK, width), lambda i: (0, i, 0))


def _row_spec(width):
    return pl.BlockSpec((ROW_BLOCK, width), lambda i: (i, 0))


def _full_spec(shape):
    return pl.BlockSpec(shape, lambda i: (0,) * len(shape))


_PAIR128 = jax.ShapeDtypeStruct((2, N, HALF), jnp.float32)

_prep = pl.pallas_call(
    _prep_body,
    grid=(GRID,),
    in_specs=[_pair_spec(HALF), _row_spec(IN_DIM)],
    out_specs=_pair_spec(HALF),
    out_shape=_PAIR128,
)

_dense = pl.pallas_call(
    _dense_body,
    grid=(GRID,),
    in_specs=[
        _pair_spec(HALF),
        _pair_spec(HALF),
        _pair_spec(HALF),
        _full_spec((IN_DIM, HID_DIM)),
        _full_spec((1, HID_DIM)),
        _full_spec((HID_DIM, OUT_DIM)),
    ],
    out_specs=_pair_spec(HALF),
    out_shape=_PAIR128,
)

_final = pl.pallas_call(
    _final_body,
    grid=(GRID,),
    in_specs=[
        _pair_spec(HALF),
        _pair_spec(HALF),
        _pair_spec(HALF),
        _full_spec((1, OUT_DIM)),
    ],
    out_specs=_row_spec(OUT_DIM),
    out_shape=jax.ShapeDtypeStruct((N, OUT_DIM), jnp.float32),
)


def kernel(x, edge_index, W1, b1, W2, b2):
    src = edge_index[0]
    dst = edge_index[1]
    pad = EPAD - E
    src2d = jnp.concatenate(
        [src, jnp.zeros((pad,), jnp.int32)]).reshape(NCHUNKS, CHUNK)
    srcoff = jnp.concatenate([src2d, src2d + N], axis=0)
    dst2d = jnp.concatenate(
        [dst, jnp.full((pad,), N, jnp.int32)]).reshape(NCHUNKS, CHUNK)

    zeros128 = jnp.zeros((ROWS_PER_TILE, HALF), jnp.float32)
    ones128 = jnp.ones((CHUNK, HALF), jnp.float32)

    deg = _deg_kernel(dst2d, zeros128, ones128).reshape(2, NPAD, HALF)
    y = _prep(deg, x)
    y_flat = y.reshape(2 * N, HALF)
    z = _agg_kernel(y_flat, srcoff, dst2d, zeros128).reshape(2, NPAD, HALF)
    q = _dense(z, y, deg, W1, b1.reshape(1, HID_DIM), W2)
    q_flat = q.reshape(2 * N, HALF)
    s = _agg_kernel(q_flat, srcoff, dst2d, zeros128).reshape(2, NPAD, HALF)
    return _final(s, q, deg, b2.reshape(1, OUT_DIM))


# speedup vs baseline: 10.2267x; 1.0584x over previous
import functools

import jax
import jax.numpy as jnp
from jax import lax
from jax.experimental import pallas as pl
from jax.experimental.pallas import tpu as pltpu
from jax.experimental.pallas import tpu_sc as plsc

N = 10000
IN_DIM = 256
HID_DIM = 512
OUT_DIM = 256
E = 160000
HALF = 128

NTILES = 16
NPAD = 10112
ROWS_PER_TILE = NPAD // NTILES
EPAD = 163840
CHUNK = 128
NCHUNKS = EPAD // CHUNK
AGG_CHUNKS_PER_TILE = NCHUNKS // NTILES
DEG_CHUNKS_PER_TILE = NCHUNKS // (2 * NTILES)

ACHUNK = 80
NBUF = 4
NPHASES = 4
A_NCHUNKS = EPAD // ACHUNK
A_CHUNKS_PER_TILE = A_NCHUNKS // NTILES

ROW_BLOCK = 1000
GRID = N // ROW_BLOCK

_mesh = plsc.VectorSubcoreMesh(core_axis_name="c", subcore_axis_name="s")


@functools.partial(
    pl.kernel,
    out_type=jax.ShapeDtypeStruct((2 * NPAD, HALF), jnp.float32),
    mesh=_mesh,
    scratch_types=[
        pltpu.VMEM_SHARED((NPAD, HALF), jnp.float32),
        pltpu.VMEM((DEG_CHUNKS_PER_TILE, CHUNK), jnp.int32),
        pltpu.VMEM((CHUNK, HALF), jnp.float32),
    ],
)
def _deg_kernel(dst_hbm, zeros_hbm, ones_hbm, out_hbm, acc, idx_v, ones_v):
    cid = lax.axis_index("c")
    sid = lax.axis_index("s")
    row0 = sid * ROWS_PER_TILE
    pltpu.sync_copy(zeros_hbm, acc.at[pl.ds(row0, ROWS_PER_TILE)])
    base = (cid * NTILES + sid) * DEG_CHUNKS_PER_TILE
    pltpu.sync_copy(dst_hbm.at[pl.ds(base, DEG_CHUNKS_PER_TILE)], idx_v)
    pltpu.sync_copy(ones_hbm, ones_v)
    plsc.subcore_barrier()

    def body(g, carry):
        pltpu.sync_copy(ones_v, acc.at[idx_v.at[g]], add=True)
        return carry

    lax.fori_loop(0, DEG_CHUNKS_PER_TILE, body, 0)
    plsc.subcore_barrier()
    pltpu.sync_copy(acc.at[pl.ds(row0, ROWS_PER_TILE)],
                    out_hbm.at[pl.ds(cid * NPAD + row0, ROWS_PER_TILE)])


@functools.partial(
    pl.kernel,
    out_type=jax.ShapeDtypeStruct((2 * NPAD, HALF), jnp.float32),
    mesh=_mesh,
    scratch_types=[
        pltpu.VMEM_SHARED((NPAD, HALF), jnp.float32),
        pltpu.VMEM((A_CHUNKS_PER_TILE // NPHASES, ACHUNK), jnp.int32),
        pltpu.VMEM((A_CHUNKS_PER_TILE // NPHASES, ACHUNK), jnp.int32),
        [pltpu.VMEM((ACHUNK, HALF), jnp.float32) for _ in range(NBUF)],
        [pltpu.SemaphoreType.DMA for _ in range(NBUF)],
        [pltpu.SemaphoreType.DMA for _ in range(NBUF)],
    ],
)
def _agg_kernel(y_hbm, srcoff_hbm, dst_hbm, zeros_hbm, z_hbm,
                acc, idx_s, idx_d, rows, gsem, ssem):
    cid = lax.axis_index("c")
    sid = lax.axis_index("s")
    row0 = sid * ROWS_PER_TILE
    phase_len = A_CHUNKS_PER_TILE // NPHASES
    pltpu.sync_copy(zeros_hbm, acc.at[pl.ds(row0, ROWS_PER_TILE)])
    plsc.subcore_barrier()

    def gather(g, b):
        pltpu.async_copy(y_hbm.at[idx_s.at[g]], rows[b], gsem[b])

    def scatter(g, b):
        pltpu.async_copy(rows[b], acc.at[idx_d.at[g]], ssem[b], add=True)

    def gwait(g, b):
        pltpu.make_async_copy(y_hbm.at[idx_s.at[g]], rows[b], gsem[b]).wait()

    def swait(g, b):
        pltpu.make_async_copy(rows[b], acc.at[idx_d.at[g]], ssem[b]).wait()

    for phase in range(NPHASES):
        pltpu.sync_copy(
            srcoff_hbm.at[pl.ds(cid * A_NCHUNKS + sid * A_CHUNKS_PER_TILE
                                + phase * phase_len, phase_len)], idx_s)
        pltpu.sync_copy(
            dst_hbm.at[pl.ds(sid * A_CHUNKS_PER_TILE + phase * phase_len,
                             phase_len)], idx_d)
        for b in range(NBUF):
            gather(b, b)

        def body(i, carry):
            g = NBUF * i
            for b in range(NBUF):
                gwait(g + b, b)
                scatter(g + b, b)

            @pl.when(i < phase_len // NBUF - 1)
            def _prefetch():
                for b in range(NBUF):
                    swait(g + b, b)
                    gather(g + NBUF + b, b)

            return carry

        lax.fori_loop(0, phase_len // NBUF, body, 0)
        for b in range(NBUF):
            swait(phase_len - NBUF + b, b)
    plsc.subcore_barrier()
    pltpu.sync_copy(acc.at[pl.ds(row0, ROWS_PER_TILE)],
                    z_hbm.at[pl.ds(cid * NPAD + row0, ROWS_PER_TILE)])


def _rsqrt_deg(deg_ref):
    return lax.rsqrt(deg_ref[0, :, :1] + deg_ref[1, :, :1] + 1.0)


def _prep_body(deg_ref, x_ref, y_ref):
    d = _rsqrt_deg(deg_ref)
    y = x_ref[...] * d
    y_ref[0] = y[:, :HALF]
    y_ref[1] = y[:, HALF:]


def _dense_body(z_ref, y_ref, deg_ref, w1_ref, b1_ref, w2_ref, q_ref):
    d = _rsqrt_deg(deg_ref)
    a = jnp.concatenate(
        [z_ref[0] + y_ref[0], z_ref[1] + y_ref[1]], axis=1) * d
    h = jnp.maximum(
        jnp.dot(a, w1_ref[...], preferred_element_type=jnp.float32)
        + b1_ref[...], 0.0)
    p = jnp.dot(h, w2_ref[...], preferred_element_type=jnp.float32)
    q = p * d
    q_ref[0] = q[:, :HALF]
    q_ref[1] = q[:, HALF:]


def _final_body(s_ref, q_ref, deg_ref, b2_ref, out_ref):
    d = _rsqrt_deg(deg_ref)
    o = jnp.concatenate(
        [s_ref[0] + q_ref[0], s_ref[1] + q_ref[1]], axis=1) * d
    o = o + b2_ref[...]
    m = jnp.max(o, axis=1, keepdims=True)
    e = jnp.exp(o - m)
    out_ref[...] = (o - m) - jnp.log(jnp.sum(e, axis=1, keepdims=True))


def _pair_spec(width):
    return pl.BlockSpec((2, ROW_BLOCK, width), lambda i: (0, i, 0))


def _row_spec(width):
    return pl.BlockSpec((ROW_BLOCK, width), lambda i: (i, 0))


def _full_spec(shape):
    return pl.BlockSpec(shape, lambda i: (0,) * len(shape))


_PAIR128 = jax.ShapeDtypeStruct((2, N, HALF), jnp.float32)

_prep = pl.pallas_call(
    _prep_body,
    grid=(GRID,),
    in_specs=[_pair_spec(HALF), _row_spec(IN_DIM)],
    out_specs=_pair_spec(HALF),
    out_shape=_PAIR128,
)

_dense = pl.pallas_call(
    _dense_body,
    grid=(GRID,),
    in_specs=[
        _pair_spec(HALF),
        _pair_spec(HALF),
        _pair_spec(HALF),
        _full_spec((IN_DIM, HID_DIM)),
        _full_spec((1, HID_DIM)),
        _full_spec((HID_DIM, OUT_DIM)),
    ],
    out_specs=_pair_spec(HALF),
    out_shape=_PAIR128,
)

_final = pl.pallas_call(
    _final_body,
    grid=(GRID,),
    in_specs=[
        _pair_spec(HALF),
        _pair_spec(HALF),
        _pair_spec(HALF),
        _full_spec((1, OUT_DIM)),
    ],
    out_specs=_row_spec(OUT_DIM),
    out_shape=jax.ShapeDtypeStruct((N, OUT_DIM), jnp.float32),
)


def kernel(x, edge_index, W1, b1, W2, b2):
    src = edge_index[0]
    dst = edge_index[1]
    pad = EPAD - E
    src2d = jnp.concatenate(
        [src, jnp.zeros((pad,), jnp.int32)]).reshape(NCHUNKS, CHUNK)
    dst2d = jnp.concatenate(
        [dst, jnp.full((pad,), N, jnp.int32)]).reshape(NCHUNKS, CHUNK)
    srca = jnp.concatenate(
        [src, jnp.zeros((pad,), jnp.int32)]).reshape(A_NCHUNKS, ACHUNK)
    srcoff = jnp.concatenate([srca, srca + N], axis=0)
    dsta = jnp.concatenate(
        [dst, jnp.full((pad,), N, jnp.int32)]).reshape(A_NCHUNKS, ACHUNK)

    zeros128 = jnp.zeros((ROWS_PER_TILE, HALF), jnp.float32)
    ones128 = jnp.ones((CHUNK, HALF), jnp.float32)

    deg = _deg_kernel(dst2d, zeros128, ones128).reshape(2, NPAD, HALF)
    y = _prep(deg, x)
    y_flat = y.reshape(2 * N, HALF)
    z = _agg_kernel(y_flat, srcoff, dsta, zeros128).reshape(2, NPAD, HALF)
    q = _dense(z, y, deg, W1, b1.reshape(1, HID_DIM), W2)
    q_flat = q.reshape(2 * N, HALF)
    s = _agg_kernel(q_flat, srcoff, dsta, zeros128).reshape(2, NPAD, HALF)
    return _final(s, q, deg, b2.reshape(1, OUT_DIM))


# speedup vs baseline: 10.4009x; 1.0170x over previous
import functools

import jax
import jax.numpy as jnp
from jax import lax
from jax.experimental import pallas as pl
from jax.experimental.pallas import tpu as pltpu
from jax.experimental.pallas import tpu_sc as plsc

N = 10000
IN_DIM = 256
HID_DIM = 512
OUT_DIM = 256
E = 160000
HALF = 128

NTILES = 16
NPAD = 10112
ROWS_PER_TILE = NPAD // NTILES
EPAD = 163840
CHUNK = 128
NCHUNKS = EPAD // CHUNK
AGG_CHUNKS_PER_TILE = NCHUNKS // NTILES
DEG_CHUNKS_PER_TILE = NCHUNKS // (2 * NTILES)

ACHUNK = 80
NBUF = 4
NPHASES = 4
A_NCHUNKS = EPAD // ACHUNK
A_CHUNKS_PER_TILE = A_NCHUNKS // NTILES

ROW_BLOCK = 1000
GRID = N // ROW_BLOCK

_mesh = plsc.VectorSubcoreMesh(core_axis_name="c", subcore_axis_name="s")


@functools.partial(
    pl.kernel,
    out_type=jax.ShapeDtypeStruct((2 * NPAD, HALF), jnp.float32),
    mesh=_mesh,
    scratch_types=[
        pltpu.VMEM_SHARED((NPAD, HALF), jnp.float32),
        pltpu.VMEM((DEG_CHUNKS_PER_TILE, CHUNK), jnp.int32),
        pltpu.VMEM((CHUNK, HALF), jnp.float32),
    ],
)
def _deg_kernel(dst_hbm, zeros_hbm, ones_hbm, out_hbm, acc, idx_v, ones_v):
    cid = lax.axis_index("c")
    sid = lax.axis_index("s")
    row0 = sid * ROWS_PER_TILE
    pltpu.sync_copy(zeros_hbm, acc.at[pl.ds(row0, ROWS_PER_TILE)])
    base = (cid * NTILES + sid) * DEG_CHUNKS_PER_TILE
    pltpu.sync_copy(dst_hbm.at[pl.ds(base, DEG_CHUNKS_PER_TILE)], idx_v)
    pltpu.sync_copy(ones_hbm, ones_v)
    plsc.subcore_barrier()

    def body(g, carry):
        pltpu.sync_copy(ones_v, acc.at[idx_v.at[g]], add=True)
        return carry

    lax.fori_loop(0, DEG_CHUNKS_PER_TILE, body, 0)
    plsc.subcore_barrier()
    pltpu.sync_copy(acc.at[pl.ds(row0, ROWS_PER_TILE)],
                    out_hbm.at[pl.ds(cid * NPAD + row0, ROWS_PER_TILE)])


@functools.partial(
    pl.kernel,
    out_type=jax.ShapeDtypeStruct((2 * NPAD, HALF), jnp.float32),
    mesh=_mesh,
    scratch_types=[
        pltpu.VMEM_SHARED((NPAD, HALF), jnp.float32),
        pltpu.VMEM((A_CHUNKS_PER_TILE // NPHASES, ACHUNK), jnp.int32),
        pltpu.VMEM((A_CHUNKS_PER_TILE // NPHASES, ACHUNK), jnp.int32),
        [pltpu.VMEM((ACHUNK, HALF), jnp.float32) for _ in range(NBUF)],
        [pltpu.SemaphoreType.DMA for _ in range(NBUF)],
        [pltpu.SemaphoreType.DMA for _ in range(NBUF)],
    ],
)
def _agg_kernel(y_hbm, srcoff_hbm, dst_hbm, zeros_hbm, z_hbm,
                acc, idx_s, idx_d, rows, gsem, ssem):
    cid = lax.axis_index("c")
    sid = lax.axis_index("s")
    row0 = sid * ROWS_PER_TILE
    phase_len = A_CHUNKS_PER_TILE // NPHASES
    pltpu.sync_copy(zeros_hbm, acc.at[pl.ds(row0, ROWS_PER_TILE)])
    plsc.subcore_barrier()

    def gather(g, b):
        pltpu.async_copy(y_hbm.at[idx_s.at[g]], rows[b], gsem[b])

    def scatter(g, b):
        pltpu.async_copy(rows[b], acc.at[idx_d.at[g]], ssem[b], add=True)

    def gwait(g, b):
        pltpu.make_async_copy(y_hbm.at[idx_s.at[g]], rows[b], gsem[b]).wait()

    def swait(g, b):
        pltpu.make_async_copy(rows[b], acc.at[idx_d.at[g]], ssem[b]).wait()

    for phase in range(NPHASES):
        pltpu.sync_copy(
            srcoff_hbm.at[pl.ds(cid * A_NCHUNKS + sid * A_CHUNKS_PER_TILE
                                + phase * phase_len, phase_len)], idx_s)
        pltpu.sync_copy(
            dst_hbm.at[pl.ds(sid * A_CHUNKS_PER_TILE + phase * phase_len,
                             phase_len)], idx_d)
        for b in range(NBUF):
            gather(b, b)

        def body(i, carry):
            g = NBUF * i
            for b in range(NBUF):
                gwait(g + b, b)
                scatter(g + b, b)

            @pl.when(i < phase_len // NBUF - 1)
            def _prefetch():
                for b in range(NBUF):
                    swait(g + b, b)
                    gather(g + NBUF + b, b)

            return carry

        lax.fori_loop(0, phase_len // NBUF, body, 0)
        for b in range(NBUF):
            swait(phase_len - NBUF + b, b)
    plsc.subcore_barrier()
    pltpu.sync_copy(acc.at[pl.ds(row0, ROWS_PER_TILE)],
                    z_hbm.at[pl.ds(cid * NPAD + row0, ROWS_PER_TILE)])


def _rsqrt_deg(deg_ref):
    return lax.rsqrt(deg_ref[0, :, :1] + deg_ref[1, :, :1] + 1.0)


def _prep_body(deg_ref, x_ref, y_ref):
    d = _rsqrt_deg(deg_ref)
    y = x_ref[...] * d
    y_ref[0] = y[:, :HALF]
    y_ref[1] = y[:, HALF:]


def _dense_body(z_ref, y_ref, deg_ref, w1_ref, b1_ref, w2_ref, q_ref):
    d = _rsqrt_deg(deg_ref)
    a = jnp.concatenate(
        [z_ref[0] + y_ref[0], z_ref[1] + y_ref[1]], axis=1) * d
    h = jnp.maximum(
        jnp.dot(a.astype(jnp.bfloat16), w1_ref[...].astype(jnp.bfloat16),
                preferred_element_type=jnp.float32)
        + b1_ref[...], 0.0)
    p = jnp.dot(h.astype(jnp.bfloat16), w2_ref[...].astype(jnp.bfloat16),
                preferred_element_type=jnp.float32)
    q = p * d
    q_ref[0] = q[:, :HALF]
    q_ref[1] = q[:, HALF:]


def _final_body(s_ref, q_ref, deg_ref, b2_ref, out_ref):
    d = _rsqrt_deg(deg_ref)
    o = jnp.concatenate(
        [s_ref[0] + q_ref[0], s_ref[1] + q_ref[1]], axis=1) * d
    o = o + b2_ref[...]
    m = jnp.max(o, axis=1, keepdims=True)
    e = jnp.exp(o - m)
    out_ref[...] = (o - m) - jnp.log(jnp.sum(e, axis=1, keepdims=True))


def _pair_spec(width):
    return pl.BlockSpec((2, ROW_BLOCK, width), lambda i: (0, i, 0))


def _row_spec(width):
    return pl.BlockSpec((ROW_BLOCK, width), lambda i: (i, 0))


def _full_spec(shape):
    return pl.BlockSpec(shape, lambda i: (0,) * len(shape))


_PAIR128 = jax.ShapeDtypeStruct((2, N, HALF), jnp.float32)

_prep = pl.pallas_call(
    _prep_body,
    grid=(GRID,),
    in_specs=[_pair_spec(HALF), _row_spec(IN_DIM)],
    out_specs=_pair_spec(HALF),
    out_shape=_PAIR128,
)

_dense = pl.pallas_call(
    _dense_body,
    grid=(GRID,),
    in_specs=[
        _pair_spec(HALF),
        _pair_spec(HALF),
        _pair_spec(HALF),
        _full_spec((IN_DIM, HID_DIM)),
        _full_spec((1, HID_DIM)),
        _full_spec((HID_DIM, OUT_DIM)),
    ],
    out_specs=_pair_spec(HALF),
    out_shape=_PAIR128,
)

_final = pl.pallas_call(
    _final_body,
    grid=(GRID,),
    in_specs=[
        _pair_spec(HALF),
        _pair_spec(HALF),
        _pair_spec(HALF),
        _full_spec((1, OUT_DIM)),
    ],
    out_specs=_row_spec(OUT_DIM),
    out_shape=jax.ShapeDtypeStruct((N, OUT_DIM), jnp.float32),
)


def kernel(x, edge_index, W1, b1, W2, b2):
    src = edge_index[0]
    dst = edge_index[1]
    pad = EPAD - E
    dst2d = jnp.concatenate(
        [dst, jnp.full((pad,), N, jnp.int32)]).reshape(NCHUNKS, CHUNK)
    srca = jnp.concatenate(
        [src, jnp.zeros((pad,), jnp.int32)]).reshape(A_NCHUNKS, ACHUNK)
    srcoff = jnp.concatenate([srca, srca + N], axis=0)
    dsta = jnp.concatenate(
        [dst, jnp.full((pad,), N, jnp.int32)]).reshape(A_NCHUNKS, ACHUNK)

    zeros128 = jnp.zeros((ROWS_PER_TILE, HALF), jnp.float32)

    ones128 = jnp.ones((CHUNK, HALF), jnp.float32)
    deg = _deg_kernel(dst2d, zeros128, ones128).reshape(2, NPAD, HALF)
    y = _prep(deg, x)
    y_flat = y.reshape(2 * N, HALF)
    z = _agg_kernel(y_flat, srcoff, dsta, zeros128).reshape(2, NPAD, HALF)
    q = _dense(z, y, deg, W1, b1.reshape(1, HID_DIM), W2)
    q_flat = q.reshape(2 * N, HALF)
    s = _agg_kernel(q_flat, srcoff, dsta, zeros128).reshape(2, NPAD, HALF)
    return _final(s, q, deg, b2.reshape(1, OUT_DIM))
